# baseline (speedup 1.0000x reference)
import functools

import jax
import jax.numpy as jnp
from jax import lax
from jax.experimental import pallas as pl
from jax.experimental.pallas import tpu as pltpu
from jax.experimental.pallas import tpu_sc as plsc

Q = 512
HW = 256
C = 128
ROWS = 16

NC = 2
NS = 16
L = 16
NW = NC * NS
QPW = Q // NW

_SC_MESH = plsc.VectorSubcoreMesh(
    core_axis_name="c", subcore_axis_name="s", num_cores=NC, num_subcores=NS)


def _lane_bcast(vec, lane):
    sel = jnp.full((L,), lane, jnp.int32)
    return vec.at[sel].get(mode="promise_in_bounds")


@functools.partial(
    pl.kernel,
    out_type=jax.ShapeDtypeStruct((Q, C), jnp.float32),
    mesh=_SC_MESH,
    scratch_types=[
        pltpu.VMEM((2 * QPW,), jnp.float32),
        pltpu.VMEM((HW,), jnp.float32),
        pltpu.VMEM((HW,), jnp.float32),
        pltpu.VMEM((QPW,), jnp.int32),
        pltpu.VMEM((QPW, C), jnp.float32),
        pltpu.SemaphoreType.DMA,
    ],
)
def _sc_nn_gather(qc_hbm, gc_hbm, tab_hbm, out_hbm,
                  qc_v, gx_v, gy_v, idx_v, rows_v, sem):
    wid = lax.axis_index("s") * NC + lax.axis_index("c")
    base = wid * QPW
    pltpu.sync_copy(qc_hbm.at[pl.ds(2 * base, 2 * QPW)], qc_v)
    pltpu.sync_copy(gc_hbm.at[0], gx_v)
    pltpu.sync_copy(gc_hbm.at[1], gy_v)

    lanes = lax.iota(jnp.int32, L)
    v0 = qc_v[pl.ds(0, L)]
    v1 = qc_v[pl.ds(L, L)]
    lo = lanes < (L // 2)
    sel_lo = jnp.where(lo, 2 * lanes, 0)
    sel_hi = jnp.where(lo, 0, 2 * lanes - L)
    qxv = jnp.where(lo, v0.at[sel_lo].get(mode="promise_in_bounds"),
                    v1.at[sel_hi].get(mode="promise_in_bounds"))
    sel_lo = sel_lo + 1
    sel_hi = sel_hi + 1
    qyv = jnp.where(lo, v0.at[sel_lo].get(mode="promise_in_bounds"),
                    v1.at[sel_hi].get(mode="promise_in_bounds"))

    bd = jnp.full((L,), jnp.inf, jnp.float32)
    bi = jnp.zeros((L,), jnp.int32)
    for ck in range(HW // L):
        gxc = gx_v[pl.ds(ck * L, L)]
        gyc = gy_v[pl.ds(ck * L, L)]
        for j in range(L):
            dx = qxv - _lane_bcast(gxc, j)
            dy = qyv - _lane_bcast(gyc, j)
            d2 = dx * dx + dy * dy
            better = d2 < bd
            bi = jnp.where(better, jnp.full((L,), ck * L + j, jnp.int32), bi)
            bd = jnp.where(better, d2, bd)

    idx_v[...] = bi
    pltpu.async_copy(tab_hbm.at[idx_v], rows_v, sem).wait()
    pltpu.sync_copy(rows_v, out_hbm.at[pl.ds(base, QPW)])


def _bcast_body(feat_ref, out_ref):
    out_ref[...] = jnp.broadcast_to(feat_ref[...][None], (ROWS, Q, C))


def kernel(grid_features, grid_coords, query_coords, N):
    gf = jnp.transpose(grid_features, (0, 2, 3, 1)).reshape(HW, C)
    gc = grid_coords.reshape(2, HW)
    qc = query_coords.reshape(2 * Q)
    feat = _sc_nn_gather(qc, gc, gf)
    out = pl.pallas_call(
        _bcast_body,
        grid=(Q // ROWS,),
        in_specs=[pl.BlockSpec((Q, C), lambda i: (0, 0))],
        out_specs=pl.BlockSpec((ROWS, Q, C), lambda i: (i, 0, 0)),
        out_shape=jax.ShapeDtypeStruct((Q, Q, C), jnp.float32),
    )(feat)
    return out.reshape(1, Q, 1, Q, 1, C)


# speedup vs baseline: 1.0149x; 1.0149x over previous
import functools

import jax
import jax.numpy as jnp
from jax import lax
from jax.experimental import pallas as pl
from jax.experimental.pallas import tpu as pltpu
from jax.experimental.pallas import tpu_sc as plsc

Q = 512
HW = 256
C = 128
ROWS = 16

NC = 2
NS = 16
L = 16
NW = NC * NS
QPW = Q // NW

_SC_MESH = plsc.VectorSubcoreMesh(
    core_axis_name="c", subcore_axis_name="s", num_cores=NC, num_subcores=NS)


def _lane_bcast(vec, lane):
    sel = jnp.full((L,), lane, jnp.int32)
    return vec.at[sel].get(mode="promise_in_bounds")


@functools.partial(
    pl.kernel,
    out_type=jax.ShapeDtypeStruct((Q, C), jnp.float32),
    mesh=_SC_MESH,
    scratch_types=[
        pltpu.VMEM((2 * QPW,), jnp.float32),
        pltpu.VMEM((HW,), jnp.float32),
        pltpu.VMEM((HW,), jnp.float32),
        pltpu.VMEM((QPW,), jnp.int32),
        pltpu.VMEM((QPW, C), jnp.float32),
        pltpu.SemaphoreType.DMA,
    ],
)
def _sc_nn_gather(qc_hbm, gc_hbm, tab_hbm, out_hbm,
                  qc_v, gx_v, gy_v, idx_v, rows_v, sem):
    wid = lax.axis_index("s") * NC + lax.axis_index("c")
    base = wid * QPW
    pltpu.sync_copy(qc_hbm.at[pl.ds(2 * base, 2 * QPW)], qc_v)
    pltpu.sync_copy(gc_hbm.at[0], gx_v)
    pltpu.sync_copy(gc_hbm.at[1], gy_v)

    lanes = lax.iota(jnp.int32, L)
    v0 = qc_v[pl.ds(0, L)]
    v1 = qc_v[pl.ds(L, L)]
    lo = lanes < (L // 2)
    sel_lo = jnp.where(lo, 2 * lanes, 0)
    sel_hi = jnp.where(lo, 0, 2 * lanes - L)
    qxv = jnp.where(lo, v0.at[sel_lo].get(mode="promise_in_bounds"),
                    v1.at[sel_hi].get(mode="promise_in_bounds"))
    qyv = jnp.where(lo, v0.at[sel_lo + 1].get(mode="promise_in_bounds"),
                    v1.at[sel_hi + 1].get(mode="promise_in_bounds"))

    def chunk(ck, carry):
        bd, bi = carry
        gxc = gx_v[pl.ds(ck * L, L)]
        gyc = gy_v[pl.ds(ck * L, L)]

        def cell(j, carry2):
            bd2, bi2 = carry2
            dx = qxv - _lane_bcast(gxc, j)
            dy = qyv - _lane_bcast(gyc, j)
            d2 = dx * dx + dy * dy
            better = d2 < bd2
            cs = jnp.full((L,), ck * L + j, jnp.int32)
            return jnp.where(better, d2, bd2), jnp.where(better, cs, bi2)

        return lax.fori_loop(0, L, cell, (bd, bi))

    _, bi = lax.fori_loop(
        0, HW // L, chunk,
        (jnp.full((L,), jnp.inf, jnp.float32), jnp.zeros((L,), jnp.int32)))

    idx_v[...] = bi
    pltpu.async_copy(tab_hbm.at[idx_v], rows_v, sem).wait()
    pltpu.sync_copy(rows_v, out_hbm.at[pl.ds(base, QPW)])


def _bcast_body(feat_ref, out_ref):
    out_ref[...] = jnp.broadcast_to(feat_ref[...][None], (ROWS, Q, C))


def kernel(grid_features, grid_coords, query_coords, N):
    gf = jnp.transpose(grid_features, (0, 2, 3, 1)).reshape(HW, C)
    gc = grid_coords.reshape(2, HW)
    qc = query_coords.reshape(2 * Q)
    feat = _sc_nn_gather(qc, gc, gf)
    out = pl.pallas_call(
        _bcast_body,
        grid=(Q // ROWS,),
        in_specs=[pl.BlockSpec((Q, C), lambda i: (0, 0))],
        out_specs=pl.BlockSpec((ROWS, Q, C), lambda i: (i, 0, 0)),
        out_shape=jax.ShapeDtypeStruct((Q, Q, C), jnp.float32),
    )(feat)
    return out.reshape(1, Q, 1, Q, 1, C)


# speedup vs baseline: 1.0333x; 1.0182x over previous
import functools

import jax
import jax.numpy as jnp
from jax import lax
from jax.experimental import pallas as pl
from jax.experimental.pallas import tpu as pltpu
from jax.experimental.pallas import tpu_sc as plsc

Q = 512
HW = 256
C = 128
ROWS = 16

NC = 2
NS = 16
L = 16
NW = NC * NS
QPW = Q // NW

_SC_MESH = plsc.VectorSubcoreMesh(
    core_axis_name="c", subcore_axis_name="s", num_cores=NC, num_subcores=NS)


def _lane_bcast(vec, lane):
    sel = jnp.full((L,), lane, jnp.int32)
    return vec.at[sel].get(mode="promise_in_bounds")


@functools.partial(
    pl.kernel,
    out_type=jax.ShapeDtypeStruct((Q,), jnp.int32),
    mesh=_SC_MESH,
    scratch_types=[
        pltpu.VMEM((2 * QPW,), jnp.float32),
        pltpu.VMEM((HW,), jnp.float32),
        pltpu.VMEM((HW,), jnp.float32),
        pltpu.VMEM((QPW,), jnp.int32),
        pltpu.SemaphoreType.DMA,
    ],
)
def _sc_nn_search(qc_hbm, gc_hbm, out_hbm, qc_v, gx_v, gy_v, idx_v, sem):
    wid = lax.axis_index("s") * NC + lax.axis_index("c")
    base = wid * QPW
    cp_q = pltpu.make_async_copy(qc_hbm.at[pl.ds(2 * base, 2 * QPW)], qc_v, sem)
    cp_x = pltpu.make_async_copy(gc_hbm.at[0], gx_v, sem)
    cp_y = pltpu.make_async_copy(gc_hbm.at[1], gy_v, sem)
    cp_q.start()
    cp_x.start()
    cp_y.start()
    cp_q.wait()
    cp_x.wait()
    cp_y.wait()

    lanes = lax.iota(jnp.int32, L)
    v0 = qc_v[pl.ds(0, L)]
    v1 = qc_v[pl.ds(L, L)]
    lo = lanes < (L // 2)
    sel_lo = jnp.where(lo, 2 * lanes, 0)
    sel_hi = jnp.where(lo, 0, 2 * lanes - L)
    qxv = jnp.where(lo, v0.at[sel_lo].get(mode="promise_in_bounds"),
                    v1.at[sel_hi].get(mode="promise_in_bounds"))
    qyv = jnp.where(lo, v0.at[sel_lo + 1].get(mode="promise_in_bounds"),
                    v1.at[sel_hi + 1].get(mode="promise_in_bounds"))

    def chunk(ck, carry):
        bd, bi = carry
        gxc = gx_v[pl.ds(ck * L, L)]
        gyc = gy_v[pl.ds(ck * L, L)]
        for j in range(L):
            dx = qxv - _lane_bcast(gxc, j)
            dy = qyv - _lane_bcast(gyc, j)
            d2 = dx * dx + dy * dy
            better = d2 < bd
            cs = jnp.full((L,), ck * L + j, jnp.int32)
            bd = jnp.where(better, d2, bd)
            bi = jnp.where(better, cs, bi)
        return bd, bi

    _, bi = lax.fori_loop(
        0, HW // L, chunk,
        (jnp.full((L,), jnp.inf, jnp.float32), jnp.zeros((L,), jnp.int32)))

    idx_v[...] = bi
    pltpu.sync_copy(idx_v, out_hbm.at[pl.ds(base, QPW)])


def _bcast_body(idx_ref, gf_ref, out_ref, feat_ref):
    @pl.when(pl.program_id(0) == 0)
    def _gather():
        onehot_t = (idx_ref[...][None, :] == jax.lax.broadcasted_iota(
            jnp.int32, (HW, Q), 0)).astype(jnp.float32)
        feat_ref[...] = lax.dot_general(
            onehot_t, gf_ref[...], (((0,), (1,)), ((), ())),
            preferred_element_type=jnp.float32,
            precision=jax.lax.Precision.HIGHEST)
    out_ref[...] = jnp.broadcast_to(feat_ref[...][None], (ROWS, Q, C))


def kernel(grid_features, grid_coords, query_coords, N):
    gf = grid_features.reshape(C, HW)
    gc = grid_coords.reshape(2, HW)
    qc = query_coords.reshape(2 * Q)
    idx = _sc_nn_search(qc, gc)
    out = pl.pallas_call(
        _bcast_body,
        grid=(Q // ROWS,),
        in_specs=[
            pl.BlockSpec((Q,), lambda i: (0,)),
            pl.BlockSpec((C, HW), lambda i: (0, 0)),
        ],
        out_specs=pl.BlockSpec((ROWS, Q, C), lambda i: (i, 0, 0)),
        out_shape=jax.ShapeDtypeStruct((Q, Q, C), jnp.float32),
        scratch_shapes=[pltpu.VMEM((Q, C), jnp.float32)],
    )(idx, gf)
    return out.reshape(1, Q, 1, Q, 1, C)


# speedup vs baseline: 1.0409x; 1.0073x over previous
import functools

import jax
import jax.numpy as jnp
from jax import lax
from jax.experimental import pallas as pl
from jax.experimental.pallas import tpu as pltpu
from jax.experimental.pallas import tpu_sc as plsc

Q = 512
HW = 256
C = 128
ROWS = 16

NC = 2
NS = 16
L = 16
NW = NC * NS
QPW = Q // NW

_SC_MESH = plsc.VectorSubcoreMesh(
    core_axis_name="c", subcore_axis_name="s", num_cores=NC, num_subcores=NS)


def _lane_bcast(vec, lane):
    sel = jnp.full((L,), lane, jnp.int32)
    return vec.at[sel].get(mode="promise_in_bounds")


@functools.partial(
    pl.kernel,
    out_type=jax.ShapeDtypeStruct((Q,), jnp.int32),
    mesh=_SC_MESH,
    scratch_types=[
        pltpu.VMEM((2 * QPW,), jnp.float32),
        pltpu.VMEM((HW,), jnp.float32),
        pltpu.VMEM((HW,), jnp.float32),
        pltpu.VMEM((QPW,), jnp.int32),
        pltpu.SemaphoreType.DMA,
    ],
)
def _sc_nn_search(coords_hbm, out_hbm, qc_v, gx_v, gy_v, idx_v, sem):
    wid = lax.axis_index("s") * NC + lax.axis_index("c")
    base = wid * QPW
    cp_q = pltpu.make_async_copy(
        coords_hbm.at[pl.ds(2 * base, 2 * QPW)], qc_v, sem)
    cp_x = pltpu.make_async_copy(coords_hbm.at[pl.ds(2 * Q, HW)], gx_v, sem)
    cp_y = pltpu.make_async_copy(
        coords_hbm.at[pl.ds(2 * Q + HW, HW)], gy_v, sem)
    cp_q.start()
    cp_x.start()
    cp_y.start()
    cp_q.wait()
    cp_x.wait()
    cp_y.wait()

    lanes = lax.iota(jnp.int32, L)
    v0 = qc_v[pl.ds(0, L)]
    v1 = qc_v[pl.ds(L, L)]
    lo = lanes < (L // 2)
    sel_lo = jnp.where(lo, 2 * lanes, 0)
    sel_hi = jnp.where(lo, 0, 2 * lanes - L)
    qxv = jnp.where(lo, v0.at[sel_lo].get(mode="promise_in_bounds"),
                    v1.at[sel_hi].get(mode="promise_in_bounds"))
    qyv = jnp.where(lo, v0.at[sel_lo + 1].get(mode="promise_in_bounds"),
                    v1.at[sel_hi + 1].get(mode="promise_in_bounds"))

    def chunk(ck, carry):
        bd, bi = carry
        gxc = gx_v[pl.ds(ck * L, L)]
        gyc = gy_v[pl.ds(ck * L, L)]
        for j in range(L):
            dx = qxv - _lane_bcast(gxc, j)
            dy = qyv - _lane_bcast(gyc, j)
            d2 = dx * dx + dy * dy
            better = d2 < bd
            cs = jnp.full((L,), ck * L + j, jnp.int32)
            bd = jnp.where(better, d2, bd)
            bi = jnp.where(better, cs, bi)
        return bd, bi

    _, bi = lax.fori_loop(
        0, HW // L, chunk,
        (jnp.full((L,), jnp.inf, jnp.float32), jnp.zeros((L,), jnp.int32)))

    idx_v[...] = bi
    pltpu.sync_copy(idx_v, out_hbm.at[pl.ds(base, QPW)])


def _bcast_body(idx_ref, gf_ref, out_ref, feat_ref):
    @pl.when(pl.program_id(0) == 0)
    def _gather():
        onehot_t = (idx_ref[...][None, :] == jax.lax.broadcasted_iota(
            jnp.int32, (HW, Q), 0)).astype(jnp.float32)
        feat_ref[...] = lax.dot_general(
            onehot_t, gf_ref[...], (((0,), (1,)), ((), ())),
            preferred_element_type=jnp.float32,
            precision=jax.lax.Precision.HIGHEST)
    out_ref[...] = jnp.broadcast_to(feat_ref[...][None], (ROWS, Q, C))


def kernel(grid_features, grid_coords, query_coords, N):
    gf = grid_features.reshape(C, HW)
    coords = jnp.concatenate(
        [query_coords.reshape(2 * Q), grid_coords.reshape(2 * HW)])
    idx = _sc_nn_search(coords)
    out = pl.pallas_call(
        _bcast_body,
        grid=(Q // ROWS,),
        in_specs=[
            pl.BlockSpec((Q,), lambda i: (0,)),
            pl.BlockSpec((C, HW), lambda i: (0, 0)),
        ],
        out_specs=pl.BlockSpec((ROWS, Q, C), lambda i: (i, 0, 0)),
        out_shape=jax.ShapeDtypeStruct((Q, Q, C), jnp.float32),
        scratch_shapes=[pltpu.VMEM((Q, C), jnp.float32)],
    )(idx, gf)
    return out.reshape(1, Q, 1, Q, 1, C)
